# baseline (speedup 1.0000x reference)
import functools
import jax
import jax.numpy as jnp
from jax import lax
from jax.experimental import pallas as pl
from jax.experimental.pallas import tpu as pltpu

_EPS = 0.01
_SIZE = 256
_NPIX = _SIZE * _SIZE
_CH = 64
_NG = 4
_BLK = 2048
_NBLK = _NPIX // _BLK


def _bf(x):
    return x.astype(jnp.bfloat16).astype(jnp.float32)


def _mat4_scalars(ref):
    return [[_bf(ref[0, i, j]) for j in range(4)] for i in range(4)]


def _matmul4(a, b):
    return [[sum(a[i][k] * b[k][j] for k in range(4)) for j in range(4)]
            for i in range(4)]


def _stage_a_body(K_ref, Kinv_ref, RTi1_ref, RT2_ref, pred_ref, alpha_ref,
                  src_ref, pix_ref, wden_ref, wf_ref):
    j = pl.program_id(1)
    Km = _mat4_scalars(K_ref)
    Kinv = _mat4_scalars(Kinv_ref)
    RT = [[_bf(e) for e in row]
          for row in _matmul4(_mat4_scalars(RT2_ref), _mat4_scalars(RTi1_ref))]

    idx = j * _BLK + lax.broadcasted_iota(jnp.int32, (1, _BLK), 1)
    cc = (idx & (_SIZE - 1)).astype(jnp.float32)
    rr = (idx >> 8).astype(jnp.float32)
    X = cc / (_SIZE - 1.0) * 2.0 - 1.0
    Y = rr / (_SIZE - 1.0) * 2.0 - 1.0
    d = pred_ref[0]
    p = [_bf(X * d), _bf(-Y * d), _bf(-d), None]

    def xform(m, vec, rows):
        out = []
        for i in rows:
            acc = m[i][0] * vec[0] + m[i][1] * vec[1] + m[i][2] * vec[2]
            acc = acc + (m[i][3] if vec[3] is None else m[i][3] * vec[3])
            out.append(acc)
        return out

    cam = xform(Kinv, p, range(4))
    wrld = xform(RT, [_bf(e) for e in cam], range(4))
    u, v, z = xform(Km, [_bf(e) for e in wrld], range(3))
    mask = jnp.abs(z) < _EPS
    zc = jnp.where(mask, _EPS, z)
    sx = jnp.where(mask, -10.0, u / (-zc))
    sy = jnp.where(mask, -10.0, v / (-zc)) * -1.0
    valid = (jnp.abs(sx) <= 1.0) & (jnp.abs(sy) <= 1.0)
    px = jnp.clip((sx + 1.0) * 0.5 * (_SIZE - 1), 0, _SIZE - 1).astype(jnp.int32)
    py = jnp.clip((1.0 - sy) * 0.5 * (_SIZE - 1), 0, _SIZE - 1).astype(jnp.int32)
    pix = py * _SIZE + px
    w = alpha_ref[0] * valid.astype(jnp.float32)
    pix_ref[0] = pix
    wden_ref[0] = w
    wf_ref[0, 0] = jnp.transpose(src_ref[0, 0] * w)


def _stage_a(K, Kinv, RTi1, RT2, pred, alpha, src4):
    smem4 = lambda: pl.BlockSpec((1, 4, 4), lambda b, j, g: (b, 0, 0),
                                 memory_space=pltpu.SMEM)
    grid = (4, _NBLK, _NG)
    return pl.pallas_call(
        _stage_a_body,
        grid=grid,
        in_specs=[
            smem4(), smem4(), smem4(), smem4(),
            pl.BlockSpec((1, 1, _BLK), lambda b, j, g: (b * _NBLK + j, 0, 0)),
            pl.BlockSpec((1, 1, _BLK), lambda b, j, g: (b * _NBLK + j, 0, 0)),
            pl.BlockSpec((1, 1, 16, _BLK), lambda b, j, g: (b, g, 0, j)),
        ],
        out_specs=[
            pl.BlockSpec((1, 1, _BLK), lambda b, j, g: (b * _NBLK + j, 0, 0)),
            pl.BlockSpec((1, 1, _BLK), lambda b, j, g: (b * _NBLK + j, 0, 0)),
            pl.BlockSpec((1, 1, _BLK, 16), lambda b, j, g: (b, g, j, 0)),
        ],
        out_shape=[
            jax.ShapeDtypeStruct((4 * _NBLK, 1, _BLK), jnp.int32),
            jax.ShapeDtypeStruct((4 * _NBLK, 1, _BLK), jnp.float32),
            jax.ShapeDtypeStruct((4, _NG, _NPIX, 16), jnp.float32),
        ],
    )(K, Kinv, RTi1, RT2, pred, alpha, src4)


def _stage_c_body(num_ref, den_ref, out_ref):
    nm = num_ref[0, 0]
    dn = den_ref[0]
    out_ref[0, 0] = jnp.transpose(nm) / (dn + 1e-8)


def _stage_c(num, den):
    grid = (4, _NG, _NBLK)
    return pl.pallas_call(
        _stage_c_body,
        grid=grid,
        in_specs=[
            pl.BlockSpec((1, 1, _BLK, 16), lambda b, g, j: (b, g, j, 0)),
            pl.BlockSpec((1, 1, _BLK), lambda b, g, j: (b * _NBLK + j, 0, 0)),
        ],
        out_specs=pl.BlockSpec((1, 1, 16, _BLK), lambda b, g, j: (b, g, 0, j)),
        out_shape=jax.ShapeDtypeStruct((4, _NG, 16, _NPIX), jnp.float32),
    )(num, den)


def _scatter_host(pix, wden, wf):
    def one(pixb, wb, wfb):
        den = jnp.zeros((_NPIX,), jnp.float32).at[pixb].add(wb)
        num = jnp.zeros((_NG, _NPIX, 16), jnp.float32).at[:, pixb].add(
            wfb, indices_are_sorted=False)
        return num, den
    return jax.vmap(one)(pix, wden, wf)


def kernel(alphas, src, pred_pts, K, K_inv, RT_cam1, RTinv_cam1, RT_cam2,
           RTinv_cam2):
    bs, c, h, w = src.shape
    pred = pred_pts.reshape(bs, _NPIX)
    alpha = alphas.reshape(bs, _NPIX)
    src4 = src.reshape(bs, _NG, 16, _NPIX)
    pix3, wden3, wf = _stage_a(K, K_inv, RTinv_cam1, RT_cam2,
                               pred.reshape(bs * _NBLK, 1, _BLK),
                               alpha.reshape(bs * _NBLK, 1, _BLK), src4)
    pix = pix3.reshape(bs, _NPIX)
    wden = wden3.reshape(bs, _NPIX)
    num, den = _scatter_host(pix, wden, wf)
    out4 = _stage_c(num, den.reshape(bs * _NBLK, 1, _BLK))
    return out4.reshape(bs, _CH, _SIZE, _SIZE)


# speedup vs baseline: 7.6617x; 7.6617x over previous
import functools
import jax
import jax.numpy as jnp
from jax import lax
from jax.experimental import pallas as pl
from jax.experimental.pallas import tpu as pltpu
from jax.experimental.pallas import tpu_sc as plsc

_EPS = 0.01
_SIZE = 256
_NPIX = _SIZE * _SIZE
_CH = 64
_NGRP = 8
_GW = 8
_BLK = 2048
_NBLK = _NPIX // _BLK


def _bf(x):
    return x.astype(jnp.bfloat16).astype(jnp.float32)


def _mat4_scalars(ref):
    return [[_bf(ref[0, i, j]) for j in range(4)] for i in range(4)]


def _matmul4(a, b):
    return [[sum(a[i][k] * b[k][j] for k in range(4)) for j in range(4)]
            for i in range(4)]


def _stage_a_body(K_ref, Kinv_ref, RTi1_ref, RT2_ref, pred_ref, alpha_ref,
                  src_ref, pix_ref, wden_ref, wf_ref):
    j = pl.program_id(1)
    Km = _mat4_scalars(K_ref)
    Kinv = _mat4_scalars(Kinv_ref)
    RT = [[_bf(e) for e in row]
          for row in _matmul4(_mat4_scalars(RT2_ref), _mat4_scalars(RTi1_ref))]

    idx = j * _BLK + lax.broadcasted_iota(jnp.int32, (1, _BLK), 1)
    cc = (idx & (_SIZE - 1)).astype(jnp.float32)
    rr = (idx >> 8).astype(jnp.float32)
    X = cc / (_SIZE - 1.0) * 2.0 - 1.0
    Y = rr / (_SIZE - 1.0) * 2.0 - 1.0
    d = pred_ref[0]
    p = [_bf(X * d), _bf(-Y * d), _bf(-d), None]

    def xform(m, vec, rows):
        out = []
        for i in rows:
            acc = m[i][0] * vec[0] + m[i][1] * vec[1] + m[i][2] * vec[2]
            acc = acc + (m[i][3] if vec[3] is None else m[i][3] * vec[3])
            out.append(acc)
        return out

    cam = xform(Kinv, p, range(4))
    wrld = xform(RT, [_bf(e) for e in cam], range(4))
    u, v, z = xform(Km, [_bf(e) for e in wrld], range(3))
    mask = jnp.abs(z) < _EPS
    zc = jnp.where(mask, _EPS, z)
    sx = jnp.where(mask, -10.0, u / (-zc))
    sy = jnp.where(mask, -10.0, v / (-zc)) * -1.0
    valid = (jnp.abs(sx) <= 1.0) & (jnp.abs(sy) <= 1.0)
    px = jnp.clip((sx + 1.0) * 0.5 * (_SIZE - 1), 0, _SIZE - 1).astype(jnp.int32)
    py = jnp.clip((1.0 - sy) * 0.5 * (_SIZE - 1), 0, _SIZE - 1).astype(jnp.int32)
    pix = py * _SIZE + px
    w = alpha_ref[0] * valid.astype(jnp.float32)
    pix_ref[0] = pix
    wden_ref[0] = w
    wf_ref[0, 0] = jnp.transpose(src_ref[0, 0] * w)


def _stage_a(K, Kinv, RTi1, RT2, pred, alpha, src4):
    smem4 = lambda: pl.BlockSpec((1, 4, 4), lambda b, j, g: (b, 0, 0),
                                 memory_space=pltpu.SMEM)
    grid = (4, _NBLK, _NGRP)
    return pl.pallas_call(
        _stage_a_body,
        grid=grid,
        in_specs=[
            smem4(), smem4(), smem4(), smem4(),
            pl.BlockSpec((1, 1, _BLK), lambda b, j, g: (b * _NBLK + j, 0, 0)),
            pl.BlockSpec((1, 1, _BLK), lambda b, j, g: (b * _NBLK + j, 0, 0)),
            pl.BlockSpec((1, 1, _GW, _BLK), lambda b, j, g: (b, g, 0, j)),
        ],
        out_specs=[
            pl.BlockSpec((1, 1, _BLK), lambda b, j, g: (b * _NBLK + j, 0, 0)),
            pl.BlockSpec((1, 1, _BLK), lambda b, j, g: (b * _NBLK + j, 0, 0)),
            pl.BlockSpec((1, 1, _BLK, _GW), lambda b, j, g: (b, g, j, 0)),
        ],
        out_shape=[
            jax.ShapeDtypeStruct((4 * _NBLK, 1, _BLK), jnp.int32),
            jax.ShapeDtypeStruct((4 * _NBLK, 1, _BLK), jnp.float32),
            jax.ShapeDtypeStruct((4, _NGRP, _NPIX, _GW), jnp.float32),
        ],
    )(K, Kinv, RTi1, RT2, pred, alpha, src4)


def _stage_c_body(num_ref, den_ref, out_ref):
    nm = num_ref[0, 0]
    dn = den_ref[0]
    out_ref[0, 0] = jnp.transpose(nm) / (dn + 1e-8)


def _stage_c(num, den):
    grid = (4, _NGRP, _NBLK)
    return pl.pallas_call(
        _stage_c_body,
        grid=grid,
        in_specs=[
            pl.BlockSpec((1, 1, _BLK, _GW), lambda b, g, j: (b, g, j, 0)),
            pl.BlockSpec((1, 1, _BLK), lambda b, g, j: (b * _NBLK + j, 0, 0)),
        ],
        out_specs=pl.BlockSpec((1, 1, _GW, _BLK), lambda b, g, j: (b, g, 0, j)),
        out_shape=jax.ShapeDtypeStruct((4, _NGRP, _GW, _NPIX), jnp.float32),
    )(num, den)


_PPT = _NPIX // 16
_NQ = _PPT // 128


def _stage_b(wf, pix, wden):
    z2d = jnp.zeros((1024, _GW), jnp.float32)
    z1d = jnp.zeros((_PPT,), jnp.float32)
    mesh = plsc.VectorSubcoreMesh(core_axis_name="c", subcore_axis_name="s")

    @functools.partial(
        pl.kernel,
        out_type=[jax.ShapeDtypeStruct((4 * _NGRP, _NPIX, _GW), jnp.float32),
                  jax.ShapeDtypeStruct((4, _NPIX), jnp.float32)],
        mesh=mesh,
        compiler_params=pltpu.CompilerParams(use_tc_tiling_on_sc=False),
        scratch_types=[
            pltpu.VMEM((_NQ, 128), jnp.int32),
            pltpu.VMEM((_NQ, 128, _GW), jnp.float32),
            pltpu.VMEM((_NQ, 128), jnp.float32),
            pltpu.VMEM((1024, _GW), jnp.float32),
            pltpu.VMEM((_PPT,), jnp.float32),
            pltpu.VMEM_SHARED((_NPIX, _GW), jnp.float32),
            pltpu.VMEM_SHARED((_NPIX,), jnp.float32),
        ],
    )
    def sc_kernel(wf_hbm, pix_hbm, wden_hbm, z2d_hbm, z1d_hbm,
                  num_hbm, den_hbm,
                  idx_v, rows_v, wbuf_v, zrow_v, zden_v, acc_sh, dacc_sh):
        c = lax.axis_index("c")
        s = lax.axis_index("s")
        base = s * _PPT
        pltpu.sync_copy(z2d_hbm, zrow_v)
        pltpu.sync_copy(z1d_hbm, zden_v)

        @pl.loop(0, 2 * _NGRP)
        def _feature_round(r):
            page = c * 2 * _NGRP + r
            b = page // _NGRP
            for q in range(4):
                pltpu.sync_copy(zrow_v,
                                acc_sh.at[pl.ds(base + q * 1024, 1024), :])
            plsc.subcore_barrier()
            pltpu.sync_copy(pix_hbm.at[b, pl.ds(s * _NQ, _NQ), :], idx_v)
            pltpu.sync_copy(wf_hbm.at[page, pl.ds(s * _NQ, _NQ), :, :], rows_v)

            @pl.loop(0, _NQ)
            def _scat(q):
                pltpu.sync_copy(rows_v.at[q], acc_sh.at[idx_v.at[q]],
                                add=True)

            plsc.subcore_barrier()
            pltpu.sync_copy(acc_sh.at[pl.ds(base, _PPT), :],
                            num_hbm.at[page, pl.ds(base, _PPT), :])

        @pl.loop(0, 2)
        def _den_round(rb):
            b = c * 2 + rb
            pltpu.sync_copy(zden_v, dacc_sh.at[pl.ds(base, _PPT)])
            plsc.subcore_barrier()
            pltpu.sync_copy(pix_hbm.at[b, pl.ds(s * _NQ, _NQ), :], idx_v)
            pltpu.sync_copy(wden_hbm.at[b, pl.ds(s * _NQ, _NQ), :], wbuf_v)

            @pl.loop(0, _NQ)
            def _scat(q):
                pltpu.sync_copy(wbuf_v.at[q], dacc_sh.at[idx_v.at[q]],
                                add=True)

            plsc.subcore_barrier()
            pltpu.sync_copy(dacc_sh.at[pl.ds(base, _PPT)],
                            den_hbm.at[b, pl.ds(base, _PPT)])

    num, den = sc_kernel(wf.reshape(4 * _NGRP, 512, 128, _GW),
                         pix.reshape(4, 512, 128),
                         wden.reshape(4, 512, 128), z2d, z1d)
    return num.reshape(4, _NGRP, _NPIX, _GW), den


def kernel(alphas, src, pred_pts, K, K_inv, RT_cam1, RTinv_cam1, RT_cam2,
           RTinv_cam2):
    bs, c, h, w = src.shape
    pred = pred_pts.reshape(bs, _NPIX)
    alpha = alphas.reshape(bs, _NPIX)
    src4 = src.reshape(bs, _NGRP, _GW, _NPIX)
    pix3, wden3, wf = _stage_a(K, K_inv, RTinv_cam1, RT_cam2,
                               pred.reshape(bs * _NBLK, 1, _BLK),
                               alpha.reshape(bs * _NBLK, 1, _BLK), src4)
    num, den = _stage_b(wf, pix3, wden3)
    out4 = _stage_c(num, den.reshape(bs * _NBLK, 1, _BLK))
    return out4.reshape(bs, _CH, _SIZE, _SIZE)


# speedup vs baseline: 8.6013x; 1.1226x over previous
import functools
import jax
import jax.numpy as jnp
from jax import lax
from jax.experimental import pallas as pl
from jax.experimental.pallas import tpu as pltpu
from jax.experimental.pallas import tpu_sc as plsc

_EPS = 0.01
_SIZE = 256
_NPIX = _SIZE * _SIZE
_CH = 64
_NGRP = 8
_GW = 8
_BLK = 2048
_NBLK = _NPIX // _BLK
_PBLK = 128
_NPB = _NPIX // _PBLK


def _bf(x):
    return x.astype(jnp.bfloat16).astype(jnp.float32)


def _mat4_scalars(ref):
    return [[_bf(ref[0, i, j]) for j in range(4)] for i in range(4)]


def _matmul4(a, b):
    return [[sum(a[i][k] * b[k][j] for k in range(4)) for j in range(4)]
            for i in range(4)]


def _proj_body(K_ref, Kinv_ref, RTi1_ref, RT2_ref, pred_ref, alpha_ref,
               pix_ref, wden_ref):
    j = pl.program_id(1)
    Km = _mat4_scalars(K_ref)
    Kinv = _mat4_scalars(Kinv_ref)
    RT = [[_bf(e) for e in row]
          for row in _matmul4(_mat4_scalars(RT2_ref), _mat4_scalars(RTi1_ref))]

    r2 = lax.broadcasted_iota(jnp.int32, (1, 16, 128), 1)
    c2 = lax.broadcasted_iota(jnp.int32, (1, 16, 128), 2)
    idx = j * _BLK + r2 * 128 + c2
    cc = (idx & (_SIZE - 1)).astype(jnp.float32)
    rr = (idx >> 8).astype(jnp.float32)
    X = cc / (_SIZE - 1.0) * 2.0 - 1.0
    Y = rr / (_SIZE - 1.0) * 2.0 - 1.0
    d = pred_ref[0]
    p = [_bf(X * d), _bf(-Y * d), _bf(-d), None]

    def xform(m, vec, rows):
        out = []
        for i in rows:
            acc = m[i][0] * vec[0] + m[i][1] * vec[1] + m[i][2] * vec[2]
            acc = acc + (m[i][3] if vec[3] is None else m[i][3] * vec[3])
            out.append(acc)
        return out

    cam = xform(Kinv, p, range(4))
    wrld = xform(RT, [_bf(e) for e in cam], range(4))
    u, v, z = xform(Km, [_bf(e) for e in wrld], range(3))
    mask = jnp.abs(z) < _EPS
    zc = jnp.where(mask, _EPS, z)
    sx = jnp.where(mask, -10.0, u / (-zc))
    sy = jnp.where(mask, -10.0, v / (-zc)) * -1.0
    valid = (jnp.abs(sx) <= 1.0) & (jnp.abs(sy) <= 1.0)
    px = jnp.clip((sx + 1.0) * 0.5 * (_SIZE - 1), 0, _SIZE - 1).astype(jnp.int32)
    py = jnp.clip((1.0 - sy) * 0.5 * (_SIZE - 1), 0, _SIZE - 1).astype(jnp.int32)
    pix_ref[...] = (py * _SIZE + px)[0]
    wden_ref[...] = (alpha_ref[0] * valid.astype(jnp.float32))[0]


def _stage_a1(K, Kinv, RTi1, RT2, pred, alpha):
    smem4 = lambda: pl.BlockSpec((1, 4, 4), lambda b, j: (b, 0, 0),
                                 memory_space=pltpu.SMEM)
    grid = (4, _NBLK)
    return pl.pallas_call(
        _proj_body,
        grid=grid,
        in_specs=[
            smem4(), smem4(), smem4(), smem4(),
            pl.BlockSpec((1, 16, 128), lambda b, j: (b * _NBLK + j, 0, 0)),
            pl.BlockSpec((1, 16, 128), lambda b, j: (b * _NBLK + j, 0, 0)),
        ],
        out_specs=[
            pl.BlockSpec((16, 128), lambda b, j: (b * _NBLK + j, 0)),
            pl.BlockSpec((16, 128), lambda b, j: (b * _NBLK + j, 0)),
        ],
        out_shape=[
            jax.ShapeDtypeStruct((4 * _NBLK * 16, 128), jnp.int32),
            jax.ShapeDtypeStruct((4 * _NBLK * 16, 128), jnp.float32),
        ],
    )(K, Kinv, RTi1, RT2, pred, alpha)


def _wt_body(src_ref, w_ref, wf_ref):
    ws = src_ref[0] * w_ref[0]
    wf_ref[0] = jnp.transpose(ws)


def _stage_a2(srcf, wrow):
    grid = (4, _NPB)
    return pl.pallas_call(
        _wt_body,
        grid=grid,
        in_specs=[
            pl.BlockSpec((1, _CH, _PBLK), lambda b, j: (b, 0, j)),
            pl.BlockSpec((1, 1, _PBLK), lambda b, j: (b * _NPB + j, 0, 0)),
        ],
        out_specs=pl.BlockSpec((1, _PBLK, _CH), lambda b, j: (b, j, 0)),
        out_shape=jax.ShapeDtypeStruct((4, _NPIX, _CH), jnp.float32),
    )(srcf, wrow)


def _stage_c_body(num_ref, den_ref, out_ref):
    t = num_ref[0]
    dn = jnp.transpose(den_ref[0])
    out_ref[0] = jnp.transpose(t / (dn + 1e-8))


def _stage_c(num64, den):
    grid = (4, _NPB)
    return pl.pallas_call(
        _stage_c_body,
        grid=grid,
        in_specs=[
            pl.BlockSpec((1, _PBLK, _CH), lambda b, j: (b, j, 0)),
            pl.BlockSpec((1, 1, _PBLK), lambda b, j: (b * _NPB + j, 0, 0)),
        ],
        out_specs=pl.BlockSpec((1, _CH, _PBLK), lambda b, j: (b, 0, j)),
        out_shape=jax.ShapeDtypeStruct((4, _CH, _NPIX), jnp.float32),
    )(num64, den)


_PPT = _NPIX // 16
_NQ = _PPT // 128


def _stage_b(wf64, pix, wden):
    z2d = jnp.zeros((1024, _GW), jnp.float32)
    z1d = jnp.zeros((_PPT,), jnp.float32)
    mesh = plsc.VectorSubcoreMesh(core_axis_name="c", subcore_axis_name="s")

    @functools.partial(
        pl.kernel,
        out_type=[jax.ShapeDtypeStruct((4, _NPIX, _CH), jnp.float32),
                  jax.ShapeDtypeStruct((4, _NPIX), jnp.float32)],
        mesh=mesh,
        compiler_params=pltpu.CompilerParams(use_tc_tiling_on_sc=False),
        scratch_types=[
            pltpu.VMEM((_NQ, 128), jnp.int32),
            pltpu.VMEM((_PPT, _GW), jnp.float32),
            pltpu.VMEM((_NQ, 128), jnp.float32),
            pltpu.VMEM((1024, _GW), jnp.float32),
            pltpu.VMEM((_PPT,), jnp.float32),
            pltpu.VMEM_SHARED((_NPIX, _GW), jnp.float32),
            pltpu.VMEM_SHARED((_NPIX,), jnp.float32),
        ],
    )
    def sc_kernel(wf_hbm, pix_hbm, wden_hbm, z2d_hbm, z1d_hbm,
                  num_hbm, den_hbm,
                  idx_v, rows_v, wbuf_v, zrow_v, zden_v, acc_sh, dacc_sh):
        c = lax.axis_index("c")
        s = lax.axis_index("s")
        base = s * _PPT
        pltpu.sync_copy(z2d_hbm, zrow_v)
        pltpu.sync_copy(z1d_hbm, zden_v)

        @pl.loop(0, 2 * _NGRP)
        def _feature_round(r):
            b = c * 2 + r // _NGRP
            g = r % _NGRP
            for q in range(4):
                pltpu.sync_copy(zrow_v,
                                acc_sh.at[pl.ds(base + q * 1024, 1024), :])
            plsc.subcore_barrier()
            pltpu.sync_copy(pix_hbm.at[b, pl.ds(s * _NQ, _NQ), :], idx_v)
            pltpu.sync_copy(
                wf_hbm.at[b, pl.ds(base, _PPT), pl.ds(g * _GW, _GW)], rows_v)

            @pl.loop(0, _NQ)
            def _scat(q):
                pltpu.sync_copy(rows_v.at[pl.ds(q * 128, 128), :],
                                acc_sh.at[idx_v.at[q]], add=True)

            plsc.subcore_barrier()
            pltpu.sync_copy(acc_sh.at[pl.ds(base, _PPT), :],
                            num_hbm.at[b, pl.ds(base, _PPT),
                                       pl.ds(g * _GW, _GW)])

        @pl.loop(0, 2)
        def _den_round(rb):
            b = c * 2 + rb
            pltpu.sync_copy(zden_v, dacc_sh.at[pl.ds(base, _PPT)])
            plsc.subcore_barrier()
            pltpu.sync_copy(pix_hbm.at[b, pl.ds(s * _NQ, _NQ), :], idx_v)
            pltpu.sync_copy(wden_hbm.at[b, pl.ds(s * _NQ, _NQ), :], wbuf_v)

            @pl.loop(0, _NQ)
            def _scat(q):
                pltpu.sync_copy(wbuf_v.at[q], dacc_sh.at[idx_v.at[q]],
                                add=True)

            plsc.subcore_barrier()
            pltpu.sync_copy(dacc_sh.at[pl.ds(base, _PPT)],
                            den_hbm.at[b, pl.ds(base, _PPT)])

    num64, den = sc_kernel(wf64, pix.reshape(4, 512, 128),
                           wden.reshape(4, 512, 128), z2d, z1d)
    return num64, den


def kernel(alphas, src, pred_pts, K, K_inv, RT_cam1, RTinv_cam1, RT_cam2,
           RTinv_cam2):
    bs, c, h, w = src.shape
    pred = pred_pts.reshape(bs * _NBLK, 16, 128)
    alpha = alphas.reshape(bs * _NBLK, 16, 128)
    srcf = src.reshape(bs, _CH, _NPIX)
    pix, wden = _stage_a1(K, K_inv, RTinv_cam1, RT_cam2, pred, alpha)
    wf64 = _stage_a2(srcf, wden.reshape(bs * _NPB, 1, _PBLK))
    num64, den = _stage_b(wf64, pix, wden)
    out4 = _stage_c(num64, den.reshape(bs * _NPB, 1, _PBLK))
    return out4.reshape(bs, _CH, _SIZE, _SIZE)


# speedup vs baseline: 15.6687x; 1.8217x over previous
import functools
import jax
import jax.numpy as jnp
from jax import lax
from jax.experimental import pallas as pl
from jax.experimental.pallas import tpu as pltpu
from jax.experimental.pallas import tpu_sc as plsc

_EPS = 0.01
_SIZE = 256
_NPIX = _SIZE * _SIZE
_CH = 64
_NGRP = 8
_GW = 8
_BLK = 2048
_NBLK = _NPIX // _BLK
_PBLK = 512
_NPB = _NPIX // _PBLK


def _bf(x):
    return x.astype(jnp.bfloat16).astype(jnp.float32)


def _mat4_scalars(ref):
    return [[_bf(ref[0, i, j]) for j in range(4)] for i in range(4)]


def _matmul4(a, b):
    return [[sum(a[i][k] * b[k][j] for k in range(4)) for j in range(4)]
            for i in range(4)]


def _proj_body(K_ref, Kinv_ref, RTi1_ref, RT2_ref, pred_ref, alpha_ref,
               pix_ref, wden_ref):
    j = pl.program_id(1)
    Km = _mat4_scalars(K_ref)
    Kinv = _mat4_scalars(Kinv_ref)
    RT = [[_bf(e) for e in row]
          for row in _matmul4(_mat4_scalars(RT2_ref), _mat4_scalars(RTi1_ref))]

    r2 = lax.broadcasted_iota(jnp.int32, (1, 16, 128), 1)
    c2 = lax.broadcasted_iota(jnp.int32, (1, 16, 128), 2)
    idx = j * _BLK + r2 * 128 + c2
    cc = (idx & (_SIZE - 1)).astype(jnp.float32)
    rr = (idx >> 8).astype(jnp.float32)
    X = cc / (_SIZE - 1.0) * 2.0 - 1.0
    Y = rr / (_SIZE - 1.0) * 2.0 - 1.0
    d = pred_ref[0]
    p = [_bf(X * d), _bf(-Y * d), _bf(-d), None]

    def xform(m, vec, rows):
        out = []
        for i in rows:
            acc = m[i][0] * vec[0] + m[i][1] * vec[1] + m[i][2] * vec[2]
            acc = acc + (m[i][3] if vec[3] is None else m[i][3] * vec[3])
            out.append(acc)
        return out

    cam = xform(Kinv, p, range(4))
    wrld = xform(RT, [_bf(e) for e in cam], range(4))
    u, v, z = xform(Km, [_bf(e) for e in wrld], range(3))
    mask = jnp.abs(z) < _EPS
    zc = jnp.where(mask, _EPS, z)
    sx = jnp.where(mask, -10.0, u / (-zc))
    sy = jnp.where(mask, -10.0, v / (-zc)) * -1.0
    valid = (jnp.abs(sx) <= 1.0) & (jnp.abs(sy) <= 1.0)
    px = jnp.clip((sx + 1.0) * 0.5 * (_SIZE - 1), 0, _SIZE - 1).astype(jnp.int32)
    py = jnp.clip((1.0 - sy) * 0.5 * (_SIZE - 1), 0, _SIZE - 1).astype(jnp.int32)
    pix_ref[...] = (py * _SIZE + px)[0]
    wden_ref[...] = (alpha_ref[0] * valid.astype(jnp.float32))[0]


def _stage_a1(K, Kinv, RTi1, RT2, pred, alpha):
    smem4 = lambda: pl.BlockSpec((1, 4, 4), lambda b, j: (b, 0, 0),
                                 memory_space=pltpu.SMEM)
    grid = (4, _NBLK)
    return pl.pallas_call(
        _proj_body,
        grid=grid,
        in_specs=[
            smem4(), smem4(), smem4(), smem4(),
            pl.BlockSpec((1, 16, 128), lambda b, j: (b * _NBLK + j, 0, 0)),
            pl.BlockSpec((1, 16, 128), lambda b, j: (b * _NBLK + j, 0, 0)),
        ],
        out_specs=[
            pl.BlockSpec((16, 128), lambda b, j: (b * _NBLK + j, 0)),
            pl.BlockSpec((16, 128), lambda b, j: (b * _NBLK + j, 0)),
        ],
        out_shape=[
            jax.ShapeDtypeStruct((4 * _NBLK * 16, 128), jnp.int32),
            jax.ShapeDtypeStruct((4 * _NBLK * 16, 128), jnp.float32),
        ],
    )(K, Kinv, RTi1, RT2, pred, alpha)


def _wt_body(src_ref, w_ref, wf_ref):
    ws = src_ref[0] * w_ref[0]
    wf_ref[0] = jnp.transpose(ws)


def _stage_a2(srcf, wrow):
    grid = (4, _NPB)
    return pl.pallas_call(
        _wt_body,
        grid=grid,
        in_specs=[
            pl.BlockSpec((1, _CH, _PBLK), lambda b, j: (b, 0, j)),
            pl.BlockSpec((1, 1, _PBLK), lambda b, j: (b * _NPB + j, 0, 0)),
        ],
        out_specs=pl.BlockSpec((1, _PBLK, _CH), lambda b, j: (b, j, 0)),
        out_shape=jax.ShapeDtypeStruct((4, _NPIX, _CH), jnp.float32),
    )(srcf, wrow)


def _stage_c_body(num_ref, den_ref, out_ref):
    t = num_ref[0]
    dn = jnp.transpose(den_ref[0])
    out_ref[0] = jnp.transpose(t / (dn + 1e-8))


def _stage_c(num64, den):
    grid = (4, _NPB)
    return pl.pallas_call(
        _stage_c_body,
        grid=grid,
        in_specs=[
            pl.BlockSpec((1, _PBLK, _CH), lambda b, j: (b, j, 0)),
            pl.BlockSpec((1, 1, _PBLK), lambda b, j: (b * _NPB + j, 0, 0)),
        ],
        out_specs=pl.BlockSpec((1, _CH, _PBLK), lambda b, j: (b, 0, j)),
        out_shape=jax.ShapeDtypeStruct((4, _CH, _NPIX), jnp.float32),
    )(num64, den)


_PPT = _NPIX // 16
_NQ = _PPT // 128


def _stage_b(wf64, pix, wden):
    z2d = jnp.zeros((1024, _GW), jnp.float32)
    z1d = jnp.zeros((_PPT,), jnp.float32)
    mesh = plsc.VectorSubcoreMesh(core_axis_name="c", subcore_axis_name="s")

    @functools.partial(
        pl.kernel,
        out_type=[jax.ShapeDtypeStruct((4, _NPIX, _CH), jnp.float32),
                  jax.ShapeDtypeStruct((4, _NPIX), jnp.float32)],
        mesh=mesh,
        compiler_params=pltpu.CompilerParams(use_tc_tiling_on_sc=False),
        scratch_types=[
            pltpu.VMEM((_NQ, 128), jnp.int32),
            pltpu.VMEM((_PPT, _GW), jnp.float32),
            pltpu.VMEM((_NQ, 128), jnp.float32),
            pltpu.VMEM((1024, _GW), jnp.float32),
            pltpu.VMEM((_PPT,), jnp.float32),
            pltpu.VMEM_SHARED((_NPIX, _GW), jnp.float32),
            pltpu.VMEM_SHARED((_NPIX,), jnp.float32),
            pltpu.SemaphoreType.DMA,
        ],
    )
    def sc_kernel(wf_hbm, pix_hbm, wden_hbm, z2d_hbm, z1d_hbm,
                  num_hbm, den_hbm,
                  idx_v, rows_v, wbuf_v, zrow_v, zden_v, acc_sh, dacc_sh,
                  scat_sem):
        c = lax.axis_index("c")
        s = lax.axis_index("s")
        base = s * _PPT
        pltpu.sync_copy(z2d_hbm, zrow_v)
        pltpu.sync_copy(z1d_hbm, zden_v)

        @pl.loop(0, 2 * _NGRP)
        def _feature_round(r):
            b = c * 2 + r // _NGRP
            g = r % _NGRP
            for q in range(4):
                pltpu.sync_copy(zrow_v,
                                acc_sh.at[pl.ds(base + q * 1024, 1024), :])
            plsc.subcore_barrier()
            pltpu.sync_copy(pix_hbm.at[b, pl.ds(s * _NQ, _NQ), :], idx_v)
            pltpu.sync_copy(
                wf_hbm.at[b, pl.ds(base, _PPT), pl.ds(g * _GW, _GW)], rows_v)

            @pl.loop(0, _NQ // 8)
            def _scat(qo):
                hs = [pltpu.async_copy(
                          rows_v.at[pl.ds((qo * 8 + qq) * 128, 128), :],
                          acc_sh.at[idx_v.at[qo * 8 + qq]],
                          scat_sem, add=True)
                      for qq in range(8)]
                for h in hs:
                    h.wait()

            plsc.subcore_barrier()
            pltpu.sync_copy(acc_sh.at[pl.ds(base, _PPT), :],
                            num_hbm.at[b, pl.ds(base, _PPT),
                                       pl.ds(g * _GW, _GW)])

        @pl.loop(0, 2)
        def _den_round(rb):
            b = c * 2 + rb
            pltpu.sync_copy(zden_v, dacc_sh.at[pl.ds(base, _PPT)])
            plsc.subcore_barrier()
            pltpu.sync_copy(pix_hbm.at[b, pl.ds(s * _NQ, _NQ), :], idx_v)
            pltpu.sync_copy(wden_hbm.at[b, pl.ds(s * _NQ, _NQ), :], wbuf_v)

            @pl.loop(0, _NQ // 8)
            def _scat(qo):
                hs = [pltpu.async_copy(wbuf_v.at[qo * 8 + qq],
                                       dacc_sh.at[idx_v.at[qo * 8 + qq]],
                                       scat_sem, add=True)
                      for qq in range(8)]
                for h in hs:
                    h.wait()

            plsc.subcore_barrier()
            pltpu.sync_copy(dacc_sh.at[pl.ds(base, _PPT)],
                            den_hbm.at[b, pl.ds(base, _PPT)])

    num64, den = sc_kernel(wf64, pix.reshape(4, 512, 128),
                           wden.reshape(4, 512, 128), z2d, z1d)
    return num64, den


def kernel(alphas, src, pred_pts, K, K_inv, RT_cam1, RTinv_cam1, RT_cam2,
           RTinv_cam2):
    bs, c, h, w = src.shape
    pred = pred_pts.reshape(bs * _NBLK, 16, 128)
    alpha = alphas.reshape(bs * _NBLK, 16, 128)
    srcf = src.reshape(bs, _CH, _NPIX)
    pix, wden = _stage_a1(K, K_inv, RTinv_cam1, RT_cam2, pred, alpha)
    wf64 = _stage_a2(srcf, wden.reshape(bs * _NPB, 1, _PBLK))
    num64, den = _stage_b(wf64, pix, wden)
    out4 = _stage_c(num64, den.reshape(bs * _NPB, 1, _PBLK))
    return out4.reshape(bs, _CH, _SIZE, _SIZE)


# speedup vs baseline: 16.1282x; 1.0293x over previous
import functools
import jax
import jax.numpy as jnp
from jax import lax
from jax.experimental import pallas as pl
from jax.experimental.pallas import tpu as pltpu
from jax.experimental.pallas import tpu_sc as plsc

_EPS = 0.01
_SIZE = 256
_NPIX = _SIZE * _SIZE
_CH = 64
_NGRP = 8
_GW = 8
_BLK = 2048
_NBLK = _NPIX // _BLK
_PBLK = 512
_NPB = _NPIX // _PBLK


def _bf(x):
    return x.astype(jnp.bfloat16).astype(jnp.float32)


def _mat4_scalars(ref):
    return [[_bf(ref[0, i, j]) for j in range(4)] for i in range(4)]


def _matmul4(a, b):
    return [[sum(a[i][k] * b[k][j] for k in range(4)) for j in range(4)]
            for i in range(4)]


def _proj_body(K_ref, Kinv_ref, RTi1_ref, RT2_ref, pred_ref, alpha_ref,
               pix_ref, wden_ref):
    j = pl.program_id(1)
    Km = _mat4_scalars(K_ref)
    Kinv = _mat4_scalars(Kinv_ref)
    RT = [[_bf(e) for e in row]
          for row in _matmul4(_mat4_scalars(RT2_ref), _mat4_scalars(RTi1_ref))]

    r2 = lax.broadcasted_iota(jnp.int32, (1, 16, 128), 1)
    c2 = lax.broadcasted_iota(jnp.int32, (1, 16, 128), 2)
    idx = j * _BLK + r2 * 128 + c2
    cc = (idx & (_SIZE - 1)).astype(jnp.float32)
    rr = (idx >> 8).astype(jnp.float32)
    X = cc / (_SIZE - 1.0) * 2.0 - 1.0
    Y = rr / (_SIZE - 1.0) * 2.0 - 1.0
    d = pred_ref[0]
    p = [_bf(X * d), _bf(-Y * d), _bf(-d), None]

    def xform(m, vec, rows):
        out = []
        for i in rows:
            acc = m[i][0] * vec[0] + m[i][1] * vec[1] + m[i][2] * vec[2]
            acc = acc + (m[i][3] if vec[3] is None else m[i][3] * vec[3])
            out.append(acc)
        return out

    cam = xform(Kinv, p, range(4))
    wrld = xform(RT, [_bf(e) for e in cam], range(4))
    u, v, z = xform(Km, [_bf(e) for e in wrld], range(3))
    mask = jnp.abs(z) < _EPS
    zc = jnp.where(mask, _EPS, z)
    sx = jnp.where(mask, -10.0, u / (-zc))
    sy = jnp.where(mask, -10.0, v / (-zc)) * -1.0
    valid = (jnp.abs(sx) <= 1.0) & (jnp.abs(sy) <= 1.0)
    px = jnp.clip((sx + 1.0) * 0.5 * (_SIZE - 1), 0, _SIZE - 1).astype(jnp.int32)
    py = jnp.clip((1.0 - sy) * 0.5 * (_SIZE - 1), 0, _SIZE - 1).astype(jnp.int32)
    pix_ref[...] = (py * _SIZE + px)[0]
    wden_ref[...] = (alpha_ref[0] * valid.astype(jnp.float32))[0]


def _stage_a1(K, Kinv, RTi1, RT2, pred, alpha):
    smem4 = lambda: pl.BlockSpec((1, 4, 4), lambda b, j: (b, 0, 0),
                                 memory_space=pltpu.SMEM)
    grid = (4, _NBLK)
    return pl.pallas_call(
        _proj_body,
        grid=grid,
        in_specs=[
            smem4(), smem4(), smem4(), smem4(),
            pl.BlockSpec((1, 16, 128), lambda b, j: (b * _NBLK + j, 0, 0)),
            pl.BlockSpec((1, 16, 128), lambda b, j: (b * _NBLK + j, 0, 0)),
        ],
        out_specs=[
            pl.BlockSpec((16, 128), lambda b, j: (b * _NBLK + j, 0)),
            pl.BlockSpec((16, 128), lambda b, j: (b * _NBLK + j, 0)),
        ],
        out_shape=[
            jax.ShapeDtypeStruct((4 * _NBLK * 16, 128), jnp.int32),
            jax.ShapeDtypeStruct((4 * _NBLK * 16, 128), jnp.float32),
        ],
    )(K, Kinv, RTi1, RT2, pred, alpha)


def _wt_body(src_ref, w_ref, wf_ref):
    ws = src_ref[0] * w_ref[0]
    wf_ref[0] = jnp.transpose(ws)


def _stage_a2(srcf, wrow):
    grid = (4, _NPB)
    return pl.pallas_call(
        _wt_body,
        grid=grid,
        in_specs=[
            pl.BlockSpec((1, _CH, _PBLK), lambda b, j: (b, 0, j)),
            pl.BlockSpec((1, 1, _PBLK), lambda b, j: (b * _NPB + j, 0, 0)),
        ],
        out_specs=pl.BlockSpec((1, _PBLK, _CH), lambda b, j: (b, j, 0)),
        out_shape=jax.ShapeDtypeStruct((4, _NPIX, _CH), jnp.float32),
    )(srcf, wrow)


def _stage_c_body(num_ref, den_ref, out_ref):
    t = num_ref[0]
    dn = jnp.transpose(den_ref[0])
    out_ref[0] = jnp.transpose(t / (dn + 1e-8))


def _stage_c(num64, den):
    grid = (4, _NPB)
    return pl.pallas_call(
        _stage_c_body,
        grid=grid,
        in_specs=[
            pl.BlockSpec((1, _PBLK, _CH), lambda b, j: (b, j, 0)),
            pl.BlockSpec((1, 1, _PBLK), lambda b, j: (b * _NPB + j, 0, 0)),
        ],
        out_specs=pl.BlockSpec((1, _CH, _PBLK), lambda b, j: (b, 0, j)),
        out_shape=jax.ShapeDtypeStruct((4, _CH, _NPIX), jnp.float32),
    )(num64, den)


_PPT = _NPIX // 16
_NQ = _PPT // 128


def _stage_b(wf64, pix, wden):
    z2d = jnp.zeros((1024, _GW), jnp.float32)
    z1d = jnp.zeros((_PPT,), jnp.float32)
    mesh = plsc.VectorSubcoreMesh(core_axis_name="c", subcore_axis_name="s")

    @functools.partial(
        pl.kernel,
        out_type=[jax.ShapeDtypeStruct((4, _NPIX, _CH), jnp.float32),
                  jax.ShapeDtypeStruct((4, _NPIX), jnp.float32)],
        mesh=mesh,
        compiler_params=pltpu.CompilerParams(use_tc_tiling_on_sc=False),
        scratch_types=[
            pltpu.VMEM((_NQ, 128), jnp.int32),
            pltpu.VMEM((_PPT // 2, _GW), jnp.float32),
            pltpu.VMEM((_PPT // 2, _GW), jnp.float32),
            pltpu.VMEM((_NQ, 128), jnp.float32),
            pltpu.VMEM((1024, _GW), jnp.float32),
            pltpu.VMEM((_PPT,), jnp.float32),
            pltpu.VMEM_SHARED((_NPIX, _GW), jnp.float32),
            pltpu.VMEM_SHARED((_NPIX,), jnp.float32),
            pltpu.SemaphoreType.DMA,
            pltpu.SemaphoreType.DMA,
        ],
    )
    def sc_kernel(wf_hbm, pix_hbm, wden_hbm, z2d_hbm, z1d_hbm,
                  num_hbm, den_hbm,
                  idx_v, rows0_v, rows1_v, wbuf_v, zrow_v, zden_v,
                  acc_sh, dacc_sh, scat_sem, g_sem):
        c = lax.axis_index("c")
        s = lax.axis_index("s")
        base = s * _PPT
        pltpu.sync_copy(z2d_hbm, zrow_v)
        pltpu.sync_copy(z1d_hbm, zden_v)

        @pl.loop(0, 2 * _NGRP)
        def _feature_round(r):
            b = r // 4
            g = c * 4 + r % 4
            for q in range(4):
                pltpu.sync_copy(zrow_v,
                                acc_sh.at[pl.ds(base + q * 1024, 1024), :])
            plsc.subcore_barrier()
            pltpu.sync_copy(pix_hbm.at[b, pl.ds(s * _NQ, _NQ), :], idx_v)
            half = _PPT // 2
            h0 = pltpu.async_copy(
                wf_hbm.at[b, pl.ds(base, half), pl.ds(g * _GW, _GW)],
                rows0_v, g_sem)
            h1 = pltpu.async_copy(
                wf_hbm.at[b, pl.ds(base + half, half), pl.ds(g * _GW, _GW)],
                rows1_v, g_sem)
            h0.wait()

            @pl.loop(0, 2)
            def _scat0(qo):
                hs = [pltpu.async_copy(
                          rows0_v.at[pl.ds((qo * 8 + qq) * 128, 128), :],
                          acc_sh.at[idx_v.at[qo * 8 + qq]],
                          scat_sem, add=True)
                      for qq in range(8)]
                for h in hs:
                    h.wait()

            h1.wait()

            @pl.loop(0, 2)
            def _scat1(qo):
                hs = [pltpu.async_copy(
                          rows1_v.at[pl.ds((qo * 8 + qq) * 128, 128), :],
                          acc_sh.at[idx_v.at[16 + qo * 8 + qq]],
                          scat_sem, add=True)
                      for qq in range(8)]
                for h in hs:
                    h.wait()

            plsc.subcore_barrier()
            pltpu.sync_copy(acc_sh.at[pl.ds(base, _PPT), :],
                            num_hbm.at[b, pl.ds(base, _PPT),
                                       pl.ds(g * _GW, _GW)])

        @pl.loop(0, 2)
        def _den_round(rb):
            b = c * 2 + rb
            pltpu.sync_copy(zden_v, dacc_sh.at[pl.ds(base, _PPT)])
            plsc.subcore_barrier()
            pltpu.sync_copy(pix_hbm.at[b, pl.ds(s * _NQ, _NQ), :], idx_v)
            pltpu.sync_copy(wden_hbm.at[b, pl.ds(s * _NQ, _NQ), :], wbuf_v)

            @pl.loop(0, _NQ // 8)
            def _scat(qo):
                hs = [pltpu.async_copy(wbuf_v.at[qo * 8 + qq],
                                       dacc_sh.at[idx_v.at[qo * 8 + qq]],
                                       scat_sem, add=True)
                      for qq in range(8)]
                for h in hs:
                    h.wait()

            plsc.subcore_barrier()
            pltpu.sync_copy(dacc_sh.at[pl.ds(base, _PPT)],
                            den_hbm.at[b, pl.ds(base, _PPT)])

    num64, den = sc_kernel(wf64, pix.reshape(4, 512, 128),
                           wden.reshape(4, 512, 128), z2d, z1d)
    return num64, den


def kernel(alphas, src, pred_pts, K, K_inv, RT_cam1, RTinv_cam1, RT_cam2,
           RTinv_cam2):
    bs, c, h, w = src.shape
    pred = pred_pts.reshape(bs * _NBLK, 16, 128)
    alpha = alphas.reshape(bs * _NBLK, 16, 128)
    srcf = src.reshape(bs, _CH, _NPIX)
    pix, wden = _stage_a1(K, K_inv, RTinv_cam1, RT_cam2, pred, alpha)
    wf64 = _stage_a2(srcf, wden.reshape(bs * _NPB, 1, _PBLK))
    num64, den = _stage_b(wf64, pix, wden)
    out4 = _stage_c(num64, den.reshape(bs * _NPB, 1, _PBLK))
    return out4.reshape(bs, _CH, _SIZE, _SIZE)


# speedup vs baseline: 17.5931x; 1.0908x over previous
import functools
import jax
import jax.numpy as jnp
from jax import lax
from jax.experimental import pallas as pl
from jax.experimental.pallas import tpu as pltpu
from jax.experimental.pallas import tpu_sc as plsc

_EPS = 0.01
_SIZE = 256
_NPIX = _SIZE * _SIZE
_CH = 64
_NGRP = 8
_GW = 8
_BLK = 2048
_NBLK = _NPIX // _BLK
_PBLK = 512
_NPB = _NPIX // _PBLK


def _bf(x):
    return x.astype(jnp.bfloat16).astype(jnp.float32)


def _mat4_scalars(ref):
    return [[_bf(ref[0, i, j]) for j in range(4)] for i in range(4)]


def _matmul4(a, b):
    return [[sum(a[i][k] * b[k][j] for k in range(4)) for j in range(4)]
            for i in range(4)]


def _proj_body(K_ref, Kinv_ref, RTi1_ref, RT2_ref, pred_ref, alpha_ref,
               pix_ref, wden_ref):
    j = pl.program_id(1)
    Km = _mat4_scalars(K_ref)
    Kinv = _mat4_scalars(Kinv_ref)
    RT = [[_bf(e) for e in row]
          for row in _matmul4(_mat4_scalars(RT2_ref), _mat4_scalars(RTi1_ref))]

    r2 = lax.broadcasted_iota(jnp.int32, (1, 16, 128), 1)
    c2 = lax.broadcasted_iota(jnp.int32, (1, 16, 128), 2)
    idx = j * _BLK + r2 * 128 + c2
    cc = (idx & (_SIZE - 1)).astype(jnp.float32)
    rr = (idx >> 8).astype(jnp.float32)
    X = cc / (_SIZE - 1.0) * 2.0 - 1.0
    Y = rr / (_SIZE - 1.0) * 2.0 - 1.0
    d = pred_ref[0]
    p = [_bf(X * d), _bf(-Y * d), _bf(-d), None]

    def xform(m, vec, rows):
        out = []
        for i in rows:
            acc = m[i][0] * vec[0] + m[i][1] * vec[1] + m[i][2] * vec[2]
            acc = acc + (m[i][3] if vec[3] is None else m[i][3] * vec[3])
            out.append(acc)
        return out

    cam = xform(Kinv, p, range(4))
    wrld = xform(RT, [_bf(e) for e in cam], range(4))
    u, v, z = xform(Km, [_bf(e) for e in wrld], range(3))
    mask = jnp.abs(z) < _EPS
    zc = jnp.where(mask, _EPS, z)
    sx = jnp.where(mask, -10.0, u / (-zc))
    sy = jnp.where(mask, -10.0, v / (-zc)) * -1.0
    valid = (jnp.abs(sx) <= 1.0) & (jnp.abs(sy) <= 1.0)
    px = jnp.clip((sx + 1.0) * 0.5 * (_SIZE - 1), 0, _SIZE - 1).astype(jnp.int32)
    py = jnp.clip((1.0 - sy) * 0.5 * (_SIZE - 1), 0, _SIZE - 1).astype(jnp.int32)
    pix_ref[...] = (py * _SIZE + px)[0]
    wden_ref[...] = (alpha_ref[0] * valid.astype(jnp.float32))[0]


def _stage_a1(K, Kinv, RTi1, RT2, pred, alpha):
    smem4 = lambda: pl.BlockSpec((1, 4, 4), lambda b, j: (b, 0, 0),
                                 memory_space=pltpu.SMEM)
    grid = (4, _NBLK)
    return pl.pallas_call(
        _proj_body,
        grid=grid,
        in_specs=[
            smem4(), smem4(), smem4(), smem4(),
            pl.BlockSpec((1, 16, 128), lambda b, j: (b * _NBLK + j, 0, 0)),
            pl.BlockSpec((1, 16, 128), lambda b, j: (b * _NBLK + j, 0, 0)),
        ],
        out_specs=[
            pl.BlockSpec((16, 128), lambda b, j: (b * _NBLK + j, 0)),
            pl.BlockSpec((16, 128), lambda b, j: (b * _NBLK + j, 0)),
        ],
        out_shape=[
            jax.ShapeDtypeStruct((4 * _NBLK * 16, 128), jnp.int32),
            jax.ShapeDtypeStruct((4 * _NBLK * 16, 128), jnp.float32),
        ],
    )(K, Kinv, RTi1, RT2, pred, alpha)


def _wt_body(src_ref, w_ref, wf_ref):
    ws = src_ref[0] * w_ref[0]
    wf_ref[0] = jnp.concatenate(
        [jnp.transpose(ws[:, :256]), jnp.transpose(ws[:, 256:])], axis=1)


def _stage_a2(srcf, wrow):
    grid = (4, _NPB)
    return pl.pallas_call(
        _wt_body,
        grid=grid,
        in_specs=[
            pl.BlockSpec((1, _CH, _PBLK), lambda b, j: (b, 0, j)),
            pl.BlockSpec((1, 1, _PBLK), lambda b, j: (b * _NPB + j, 0, 0)),
        ],
        out_specs=pl.BlockSpec((1, _PBLK // 2, 128), lambda b, j: (b, j, 0)),
        out_shape=jax.ShapeDtypeStruct((4, _NPIX // 2, 128), jnp.float32),
    )(srcf, wrow)


def _stage_c_body(num_ref, den_ref, out_ref):
    t = num_ref[0]
    dn = den_ref[0]
    d0 = jnp.broadcast_to(jnp.transpose(dn[:, :256]), (256, 64))
    d1 = jnp.broadcast_to(jnp.transpose(dn[:, 256:]), (256, 64))
    o = t / (jnp.concatenate([d0, d1], axis=1) + 1e-8)
    out_ref[0] = jnp.concatenate(
        [jnp.transpose(o[:, :64]), jnp.transpose(o[:, 64:])], axis=1)


def _stage_c(num64, den):
    grid = (4, _NPB)
    return pl.pallas_call(
        _stage_c_body,
        grid=grid,
        in_specs=[
            pl.BlockSpec((1, _PBLK // 2, 128), lambda b, j: (b, j, 0)),
            pl.BlockSpec((1, 1, _PBLK), lambda b, j: (b * _NPB + j, 0, 0)),
        ],
        out_specs=pl.BlockSpec((1, _CH, _PBLK), lambda b, j: (b, 0, j)),
        out_shape=jax.ShapeDtypeStruct((4, _CH, _NPIX), jnp.float32),
    )(num64, den)


_PPT = _NPIX // 16
_NQ = _PPT // 128


def _stage_b(wf64, pix, wden):
    z2d = jnp.zeros((1024, _GW), jnp.float32)
    z1d = jnp.zeros((_PPT,), jnp.float32)
    mesh = plsc.VectorSubcoreMesh(core_axis_name="c", subcore_axis_name="s")

    @functools.partial(
        pl.kernel,
        out_type=[jax.ShapeDtypeStruct((4, _NPIX // 2, 128), jnp.float32),
                  jax.ShapeDtypeStruct((4, _NPIX), jnp.float32)],
        mesh=mesh,
        compiler_params=pltpu.CompilerParams(use_tc_tiling_on_sc=False),
        scratch_types=[
            pltpu.VMEM((_NQ, 128), jnp.int32),
            pltpu.VMEM((_PPT // 2, _GW), jnp.float32),
            pltpu.VMEM((_PPT // 2, _GW), jnp.float32),
            pltpu.VMEM((_NQ, 128), jnp.float32),
            pltpu.VMEM((1024, _GW), jnp.float32),
            pltpu.VMEM((_PPT,), jnp.float32),
            pltpu.VMEM_SHARED((_NPIX, _GW), jnp.float32),
            pltpu.VMEM_SHARED((_NPIX,), jnp.float32),
            pltpu.SemaphoreType.DMA,
            pltpu.SemaphoreType.DMA,
        ],
    )
    def sc_kernel(wf_hbm, pix_hbm, wden_hbm, z2d_hbm, z1d_hbm,
                  num_hbm, den_hbm,
                  idx_v, rows0_v, rows1_v, wbuf_v, zrow_v, zden_v,
                  acc_sh, dacc_sh, scat_sem, g_sem):
        c = lax.axis_index("c")
        s = lax.axis_index("s")
        base = s * _PPT
        pltpu.sync_copy(z2d_hbm, zrow_v)
        pltpu.sync_copy(z1d_hbm, zden_v)

        @pl.loop(0, 2 * _NGRP)
        def _feature_round(r):
            b = r // 4
            g = c * 4 + r % 4
            for q in range(4):
                pltpu.sync_copy(zrow_v,
                                acc_sh.at[pl.ds(base + q * 1024, 1024), :])
            plsc.subcore_barrier()
            pltpu.sync_copy(pix_hbm.at[b, pl.ds(s * _NQ, _NQ), :], idx_v)
            rbase = s * (_PPT // 2)
            h0 = pltpu.async_copy(
                wf_hbm.at[b, pl.ds(rbase, _PPT // 2),
                          pl.ds(g * _GW, _GW)],
                rows0_v, g_sem)
            h1 = pltpu.async_copy(
                wf_hbm.at[b, pl.ds(rbase, _PPT // 2),
                          pl.ds(64 + g * _GW, _GW)],
                rows1_v, g_sem)
            h0.wait()

            @pl.loop(0, 2)
            def _scat0(jo):
                hs = []
                for k in range(4):
                    jj = jo * 4 + k
                    for t in range(2):
                        hs.append(pltpu.async_copy(
                            rows0_v.at[pl.ds(jj * 256 + t * 128, 128), :],
                            acc_sh.at[idx_v.at[jj * 4 + t]],
                            scat_sem, add=True))
                for h in hs:
                    h.wait()

            h1.wait()

            @pl.loop(0, 2)
            def _scat1(jo):
                hs = []
                for k in range(4):
                    jj = jo * 4 + k
                    for t in range(2):
                        hs.append(pltpu.async_copy(
                            rows1_v.at[pl.ds(jj * 256 + t * 128, 128), :],
                            acc_sh.at[idx_v.at[jj * 4 + 2 + t]],
                            scat_sem, add=True))
                for h in hs:
                    h.wait()

            plsc.subcore_barrier()

            @pl.loop(0, 2)
            def _drain(jo):
                hs = []
                for k in range(4):
                    jj = jo * 4 + k
                    for t in range(2):
                        hs.append(pltpu.async_copy(
                            acc_sh.at[pl.ds(base + jj * 512 + t * 256, 256),
                                      :],
                            num_hbm.at[b, pl.ds(rbase + jj * 256, 256),
                                       pl.ds(64 * t + g * _GW, _GW)],
                            scat_sem))
                for h in hs:
                    h.wait()

        @pl.loop(0, 2)
        def _den_round(rb):
            b = c * 2 + rb
            pltpu.sync_copy(zden_v, dacc_sh.at[pl.ds(base, _PPT)])
            plsc.subcore_barrier()
            pltpu.sync_copy(pix_hbm.at[b, pl.ds(s * _NQ, _NQ), :], idx_v)
            pltpu.sync_copy(wden_hbm.at[b, pl.ds(s * _NQ, _NQ), :], wbuf_v)

            @pl.loop(0, _NQ // 8)
            def _scat(qo):
                hs = [pltpu.async_copy(wbuf_v.at[qo * 8 + qq],
                                       dacc_sh.at[idx_v.at[qo * 8 + qq]],
                                       scat_sem, add=True)
                      for qq in range(8)]
                for h in hs:
                    h.wait()

            plsc.subcore_barrier()
            pltpu.sync_copy(dacc_sh.at[pl.ds(base, _PPT)],
                            den_hbm.at[b, pl.ds(base, _PPT)])

    num64, den = sc_kernel(wf64, pix.reshape(4, 512, 128),
                           wden.reshape(4, 512, 128), z2d, z1d)
    return num64, den


def kernel(alphas, src, pred_pts, K, K_inv, RT_cam1, RTinv_cam1, RT_cam2,
           RTinv_cam2):
    bs, c, h, w = src.shape
    pred = pred_pts.reshape(bs * _NBLK, 16, 128)
    alpha = alphas.reshape(bs * _NBLK, 16, 128)
    srcf = src.reshape(bs, _CH, _NPIX)
    pix, wden = _stage_a1(K, K_inv, RTinv_cam1, RT_cam2, pred, alpha)
    wf64 = _stage_a2(srcf, wden.reshape(bs * _NPB, 1, _PBLK))
    num64, den = _stage_b(wf64, pix, wden)
    out4 = _stage_c(num64, den.reshape(bs * _NPB, 1, _PBLK))
    return out4.reshape(bs, _CH, _SIZE, _SIZE)
